# baseline (device time: 133540 ns/iter reference)
import jax
import jax.numpy as jnp
from jax import lax
from jax.experimental import pallas as pl
from jax.experimental.pallas import tpu as pltpu

N_DEV = 4


def _ring_allreduce(p):
    rows, cols = p.shape

    def body(p_ref, out_ref, comm_ref, send_sems, recv_sems):
        my_pos = lax.axis_index("i")
        left = lax.rem(my_pos - 1 + N_DEV, N_DEV)
        right = lax.rem(my_pos + 1, N_DEV)

        barrier_sem = pltpu.get_barrier_semaphore()
        for nbr in (left, right):
            pl.semaphore_signal(
                barrier_sem, inc=1,
                device_id=(nbr,), device_id_type=pl.DeviceIdType.MESH,
            )
        pl.semaphore_wait(barrier_sem, 2)

        comm_ref[0, :, :] = p_ref[:, :]
        out_ref[:, :] = p_ref[:, :].astype(jnp.float32)

        for h in range(N_DEV - 1):
            rdma = pltpu.make_async_remote_copy(
                src_ref=comm_ref.at[h],
                dst_ref=comm_ref.at[h + 1],
                send_sem=send_sems.at[h],
                recv_sem=recv_sems.at[h],
                device_id=(right,),
                device_id_type=pl.DeviceIdType.MESH,
            )
            rdma.start()
            rdma.wait()
            out_ref[:, :] = out_ref[:, :] + comm_ref[h + 1, :, :].astype(
                jnp.float32
            )

    return pl.pallas_call(
        body,
        out_shape=jax.ShapeDtypeStruct((rows, cols), jnp.float32),
        in_specs=[pl.BlockSpec(memory_space=pltpu.VMEM)],
        out_specs=pl.BlockSpec(memory_space=pltpu.VMEM),
        scratch_shapes=[
            pltpu.VMEM((N_DEV, rows, cols), p.dtype),
            pltpu.SemaphoreType.DMA((N_DEV - 1,)),
            pltpu.SemaphoreType.DMA((N_DEV - 1,)),
        ],
        compiler_params=pltpu.CompilerParams(collective_id=0),
    )(p)


def kernel(x, Wq, Wo, K_ext, V_ext):
    B, Sq, Dm = x.shape
    Dh = 128
    Hq_local = Wq.shape[1] // Dh
    scale = 0.08838834764831843

    i = lax.axis_index("i")

    xb = x.astype(jnp.bfloat16)
    Q = jnp.einsum(
        "bsd,df->bsf", xb, Wq.astype(jnp.bfloat16),
        preferred_element_type=jnp.bfloat16,
    ).reshape(B, Sq, Hq_local, Dh)

    K2 = lax.dynamic_slice_in_dim(K_ext, 2 * i, 2, axis=2)
    V2 = lax.dynamic_slice_in_dim(V_ext, 2 * i, 2, axis=2)
    Kr = jnp.repeat(K2, 4, axis=2).astype(jnp.bfloat16)
    Vr = jnp.repeat(V2, 4, axis=2).astype(jnp.bfloat16)

    s = jnp.einsum(
        "bqhd,bkhd->bhqk", Q, Kr, preferred_element_type=jnp.float32
    ) * scale
    m = jnp.max(s, axis=-1, keepdims=True)
    p = jnp.exp(s - m)
    l = jnp.sum(p, axis=-1, keepdims=True)
    p = (p / l).astype(jnp.bfloat16)
    o = jnp.einsum(
        "bhqk,bkhd->bqhd", p, Vr, preferred_element_type=jnp.bfloat16
    )
    attn = o.reshape(B, Sq, Hq_local * Dh)

    partial = jnp.einsum(
        "bsf,fd->bsd", attn, Wo.astype(jnp.bfloat16),
        preferred_element_type=jnp.bfloat16,
    )

    out2d = _ring_allreduce(partial.reshape(B * Sq, Dm))
    return out2d.reshape(B, Sq, Dm)


# device time: 99058 ns/iter; 1.3481x vs baseline; 1.3481x over previous
import jax
import jax.numpy as jnp
from jax import lax
from jax.experimental import pallas as pl
from jax.experimental.pallas import tpu as pltpu

N_DEV = 4


def _ring_allreduce_bidir(p):
    rows, cols = p.shape
    half = rows // 2
    chunk = half // N_DEV

    def body(p_ref, out_ref, r_buf, st_buf, a_buf, send_sems, recv_sems):
        my = lax.axis_index("i")
        left = lax.rem(my + N_DEV - 1, N_DEV)
        right = lax.rem(my + 1, N_DEV)

        barrier_sem = pltpu.get_barrier_semaphore()
        for nbr in (left, right):
            pl.semaphore_signal(
                barrier_sem, inc=1,
                device_id=(nbr,), device_id_type=pl.DeviceIdType.MESH,
            )
        pl.semaphore_wait(barrier_sem, 2)

        peer = (right, left)
        base = (0, half)

        def crow(d, idx):
            return base[d] + lax.rem(idx + 4 * N_DEV, N_DEV) * chunk

        rdmas = []

        for s in range(N_DEV - 1):
            step_rdmas = []
            for d in range(2):
                sgn = 1 if d == 0 else -1
                if s == 0:
                    src = p_ref.at[pl.ds(crow(d, my), chunk)]
                else:
                    src = st_buf.at[d, s - 1]
                rdma = pltpu.make_async_remote_copy(
                    src_ref=src,
                    dst_ref=r_buf.at[d, s],
                    send_sem=send_sems.at[d, s],
                    recv_sem=recv_sems.at[d, s],
                    device_id=(peer[d],),
                    device_id_type=pl.DeviceIdType.MESH,
                )
                rdma.start()
                step_rdmas.append(rdma)
            for d in range(2):
                sgn = 1 if d == 0 else -1
                step_rdmas[d].wait_recv()
                st_buf[d, s] = r_buf[d, s] + p_ref[
                    pl.ds(crow(d, my - sgn * (s + 1)), chunk)
                ]
            rdmas.extend(step_rdmas)

        for d in range(2):
            sgn = 1 if d == 0 else -1
            out_ref[pl.ds(crow(d, my + sgn), chunk), :] = st_buf[
                d, N_DEV - 2
            ].astype(jnp.float32)

        for s in range(N_DEV - 1):
            step_rdmas = []
            for d in range(2):
                src = st_buf.at[d, N_DEV - 2] if s == 0 else a_buf.at[d, s - 1]
                rdma = pltpu.make_async_remote_copy(
                    src_ref=src,
                    dst_ref=a_buf.at[d, s],
                    send_sem=send_sems.at[d, N_DEV - 1 + s],
                    recv_sem=recv_sems.at[d, N_DEV - 1 + s],
                    device_id=(peer[d],),
                    device_id_type=pl.DeviceIdType.MESH,
                )
                rdma.start()
                step_rdmas.append(rdma)
            for d in range(2):
                sgn = 1 if d == 0 else -1
                step_rdmas[d].wait_recv()
                out_ref[pl.ds(crow(d, my - sgn * s), chunk), :] = a_buf[
                    d, s
                ].astype(jnp.float32)
            rdmas.extend(step_rdmas)

        for rdma in rdmas:
            rdma.wait_send()

    n_step = 2 * (N_DEV - 1)
    return pl.pallas_call(
        body,
        out_shape=jax.ShapeDtypeStruct((rows, cols), jnp.float32),
        in_specs=[pl.BlockSpec(memory_space=pltpu.VMEM)],
        out_specs=pl.BlockSpec(memory_space=pltpu.VMEM),
        scratch_shapes=[
            pltpu.VMEM((2, N_DEV - 1, chunk, cols), p.dtype),
            pltpu.VMEM((2, N_DEV - 1, chunk, cols), p.dtype),
            pltpu.VMEM((2, N_DEV - 1, chunk, cols), p.dtype),
            pltpu.SemaphoreType.DMA((2, n_step)),
            pltpu.SemaphoreType.DMA((2, n_step)),
        ],
        compiler_params=pltpu.CompilerParams(collective_id=0),
    )(p)


def kernel(x, Wq, Wo, K_ext, V_ext):
    B, Sq, Dm = x.shape
    Dh = 128
    Hq_local = Wq.shape[1] // Dh
    G = 2
    R = Hq_local // G
    scale = 0.08838834764831843

    i = lax.axis_index("i")

    xb = x.astype(jnp.bfloat16)
    Q = jnp.einsum(
        "bsd,df->bsf", xb, Wq.astype(jnp.bfloat16),
        preferred_element_type=jnp.bfloat16,
    ).reshape(B, Sq, G, R, Dh)

    K2 = lax.dynamic_slice_in_dim(K_ext, G * i, G, axis=2)
    V2 = lax.dynamic_slice_in_dim(V_ext, G * i, G, axis=2)
    K2 = K2.astype(jnp.bfloat16)
    V2 = V2.astype(jnp.bfloat16)

    s = jnp.einsum(
        "bqgrd,bkgd->bgrqk", Q, K2, preferred_element_type=jnp.float32
    ) * scale
    m = jnp.max(s, axis=-1, keepdims=True)
    p = jnp.exp(s - m).astype(jnp.bfloat16)
    l = jnp.sum(p, axis=-1, dtype=jnp.float32)
    o = jnp.einsum(
        "bgrqk,bkgd->bqgrd", p, V2, preferred_element_type=jnp.float32
    )
    o = o / l.transpose(0, 3, 1, 2)[..., None]
    attn = o.astype(jnp.bfloat16).reshape(B, Sq, Hq_local * Dh)

    partial = jnp.einsum(
        "bsf,fd->bsd", attn, Wo.astype(jnp.bfloat16),
        preferred_element_type=jnp.bfloat16,
    )

    out2d = _ring_allreduce_bidir(partial.reshape(B * Sq, Dm))
    return out2d.reshape(B, Sq, Dm)


# device time: 73342 ns/iter; 1.8208x vs baseline; 1.3506x over previous
import jax
import jax.numpy as jnp
from jax import lax
from jax.experimental import pallas as pl
from jax.experimental.pallas import tpu as pltpu

N_DEV = 4


def _ring_allreduce_bidir(p):
    rows, cols = p.shape
    half = rows // 2
    chunk = half // N_DEV

    def body(p_ref, out_ref, r_buf, st_buf, a_buf, send_sems, recv_sems):
        my = lax.axis_index("i")
        left = lax.rem(my + N_DEV - 1, N_DEV)
        right = lax.rem(my + 1, N_DEV)

        barrier_sem = pltpu.get_barrier_semaphore()
        for nbr in (left, right):
            pl.semaphore_signal(
                barrier_sem, inc=1,
                device_id=(nbr,), device_id_type=pl.DeviceIdType.MESH,
            )
        pl.semaphore_wait(barrier_sem, 2)

        peer = (right, left)
        base = (0, half)

        def crow(d, idx):
            return base[d] + lax.rem(idx + 4 * N_DEV, N_DEV) * chunk

        rdmas = []

        for s in range(N_DEV - 1):
            step_rdmas = []
            for d in range(2):
                sgn = 1 if d == 0 else -1
                if s == 0:
                    src = p_ref.at[pl.ds(crow(d, my), chunk)]
                else:
                    src = st_buf.at[d, s - 1]
                rdma = pltpu.make_async_remote_copy(
                    src_ref=src,
                    dst_ref=r_buf.at[d, s],
                    send_sem=send_sems.at[d, s],
                    recv_sem=recv_sems.at[d, s],
                    device_id=(peer[d],),
                    device_id_type=pl.DeviceIdType.MESH,
                )
                rdma.start()
                step_rdmas.append(rdma)
            for d in range(2):
                sgn = 1 if d == 0 else -1
                step_rdmas[d].wait_recv()
                st_buf[d, s] = r_buf[d, s] + p_ref[
                    pl.ds(crow(d, my - sgn * (s + 1)), chunk)
                ]
            rdmas.extend(step_rdmas)

        for d in range(2):
            sgn = 1 if d == 0 else -1
            out_ref[pl.ds(crow(d, my + sgn), chunk), :] = st_buf[
                d, N_DEV - 2
            ].astype(jnp.float32)

        for s in range(N_DEV - 1):
            step_rdmas = []
            for d in range(2):
                src = st_buf.at[d, N_DEV - 2] if s == 0 else a_buf.at[d, s - 1]
                rdma = pltpu.make_async_remote_copy(
                    src_ref=src,
                    dst_ref=a_buf.at[d, s],
                    send_sem=send_sems.at[d, N_DEV - 1 + s],
                    recv_sem=recv_sems.at[d, N_DEV - 1 + s],
                    device_id=(peer[d],),
                    device_id_type=pl.DeviceIdType.MESH,
                )
                rdma.start()
                step_rdmas.append(rdma)
            for d in range(2):
                sgn = 1 if d == 0 else -1
                step_rdmas[d].wait_recv()
                out_ref[pl.ds(crow(d, my - sgn * s), chunk), :] = a_buf[
                    d, s
                ].astype(jnp.float32)
            rdmas.extend(step_rdmas)

        for rdma in rdmas:
            rdma.wait_send()

    n_step = 2 * (N_DEV - 1)
    return pl.pallas_call(
        body,
        out_shape=jax.ShapeDtypeStruct((rows, cols), jnp.float32),
        in_specs=[pl.BlockSpec(memory_space=pltpu.VMEM)],
        out_specs=pl.BlockSpec(memory_space=pltpu.VMEM),
        scratch_shapes=[
            pltpu.VMEM((2, N_DEV - 1, chunk, cols), p.dtype),
            pltpu.VMEM((2, N_DEV - 1, chunk, cols), p.dtype),
            pltpu.VMEM((2, N_DEV - 1, chunk, cols), p.dtype),
            pltpu.SemaphoreType.DMA((2, n_step)),
            pltpu.SemaphoreType.DMA((2, n_step)),
        ],
        compiler_params=pltpu.CompilerParams(collective_id=0),
    )(p)


def _attention_pallas(xb, Wqb, K2, V2, R, Dh, scale):
    B, Sq, Dm = xb.shape
    G = K2.shape[1]
    RDh = R * Dh

    def body(x_ref, wq_ref, k_ref, v_ref, o_ref):
        xblk = x_ref[0]
        q = jnp.dot(
            xblk, wq_ref[...], preferred_element_type=jnp.float32
        ).astype(jnp.bfloat16)
        k = k_ref[0, 0]
        v = v_ref[0, 0]
        for r in range(R):
            qr = q[:, r * Dh:(r + 1) * Dh]
            s = lax.dot_general(
                qr, k, (((1,), (1,)), ((), ())),
                preferred_element_type=jnp.float32,
            ) * scale
            m = jnp.max(s, axis=1, keepdims=True)
            p = jnp.exp(s - m)
            l = jnp.sum(p, axis=1, keepdims=True)
            o = jnp.dot(
                p.astype(jnp.bfloat16), v,
                preferred_element_type=jnp.float32,
            ) / l
            o_ref[0, :, r * Dh:(r + 1) * Dh] = o.astype(jnp.bfloat16)

    return pl.pallas_call(
        body,
        grid=(B, G),
        in_specs=[
            pl.BlockSpec((1, Sq, Dm), lambda b, g: (b, 0, 0)),
            pl.BlockSpec((Dm, RDh), lambda b, g: (0, g)),
            pl.BlockSpec((1, 1, K2.shape[2], Dh), lambda b, g: (b, g, 0, 0)),
            pl.BlockSpec((1, 1, V2.shape[2], Dh), lambda b, g: (b, g, 0, 0)),
        ],
        out_specs=pl.BlockSpec((1, Sq, RDh), lambda b, g: (b, 0, g)),
        out_shape=jax.ShapeDtypeStruct((B, Sq, G * RDh), jnp.bfloat16),
    )(xb, Wqb, K2, V2)


def kernel(x, Wq, Wo, K_ext, V_ext):
    B, Sq, Dm = x.shape
    Dh = 128
    Hq_local = Wq.shape[1] // Dh
    G = 2
    R = Hq_local // G
    scale = 0.08838834764831843

    i = lax.axis_index("i")

    xb = x.astype(jnp.bfloat16)
    Wqb = Wq.astype(jnp.bfloat16)

    K2 = lax.dynamic_slice_in_dim(K_ext, G * i, G, axis=2)
    V2 = lax.dynamic_slice_in_dim(V_ext, G * i, G, axis=2)
    K2 = K2.transpose(0, 2, 1, 3).astype(jnp.bfloat16)
    V2 = V2.transpose(0, 2, 1, 3).astype(jnp.bfloat16)

    attn = _attention_pallas(xb, Wqb, K2, V2, R, Dh, scale)

    partial = jnp.einsum(
        "bsf,fd->bsd", attn, Wo.astype(jnp.bfloat16),
        preferred_element_type=jnp.bfloat16,
    )

    out2d = _ring_allreduce_bidir(partial.reshape(B * Sq, Dm))
    return out2d.reshape(B, Sq, Dm)
